# baseline (speedup 1.0000x reference)
import functools

import jax
import jax.numpy as jnp
from jax import lax
from jax.experimental import pallas as pl
from jax.experimental.pallas import tpu as pltpu
from jax.experimental.pallas import tpu_sc as plsc

N = 10000
S = 4
D = 256
E = 160000
H = 512
EMB = 256

F = 128
CH = (S * D) // F
NC = 2
NS = 16
EPT = E // NS
HN = 5120
ACCR = HN + 8
RPT = HN // NS
B = 80
NB = EPT // B
NQ = 2
LANES = 16


def _sc_agg(x_f, ei_flat, zrows):
    mesh = plsc.VectorSubcoreMesh(core_axis_name="c", subcore_axis_name="s",
                                  num_cores=NC)

    @functools.partial(
        pl.kernel,
        out_type=jax.ShapeDtypeStruct((CH, N, F), jnp.float32),
        mesh=mesh,
        scratch_types=(
            [pltpu.VMEM((EPT + NQ * B, ), jnp.int32)] * 2
            + [pltpu.VMEM((B,), jnp.int32)] * NQ
            + [pltpu.VMEM((B, F), jnp.float32)] * NQ
            + [pltpu.VMEM((RPT, F), jnp.float32)]
            + [pltpu.VMEM_SHARED((ACCR, F), jnp.float32)]
            + [pltpu.SemaphoreType.DMA] * (2 * NQ)
        ),
    )
    def k(x_hbm, ei_hbm, z_hbm, agg_hbm,
          srcall, dstall, d0, d1, r0, r1, zbuf, acc,
          g0, g1, s0, s1):
        dstv = [d0, d1]
        rows = [r0, r1]
        gsem = [g0, g1]
        ssem = [s0, s1]
        c = lax.axis_index("c")
        s = lax.axis_index("s")
        nbase = c * HN
        ebase = s * EPT
        rbase = s * RPT
        pltpu.sync_copy(z_hbm, zbuf)
        pltpu.sync_copy(ei_hbm.at[pl.ds(ebase, EPT)],
                        srcall.at[pl.ds(0, EPT)])
        pltpu.sync_copy(ei_hbm.at[pl.ds(E + ebase, EPT)],
                        dstall.at[pl.ds(0, EPT)])

        lane = lax.iota(jnp.int32, LANES)

        def comp(i, carry):
            dv = dstall[pl.ds(i * LANES, LANES)] - nbase
            ok = (dv >= 0) & (dv < HN)
            dstall[pl.ds(i * LANES, LANES)] = jnp.where(ok, dv, HN)
            sv = srcall[pl.ds(i * LANES, LANES)]
            srcall[pl.ds(i * LANES, LANES)] = jnp.where(ok, sv, 0)
            return carry

        lax.fori_loop(0, EPT // LANES, comp, 0)
        cnt = EPT

        for i in range((NQ * B) // LANES):
            off = EPT + i * LANES
            srcall[pl.ds(off, LANES)] = jnp.zeros((LANES,), jnp.int32)
            dstall[pl.ds(off, LANES)] = jnp.full((LANES,), HN, jnp.int32)

        nb4 = (EPT + NQ * B - 1) // (NQ * B)
        end = ((cnt + NQ * B - 1) // (NQ * B)) * (NQ * B)

        for g in range(CH):
            pltpu.sync_copy(zbuf, acc.at[pl.ds(rbase, RPT)])
            plsc.subcore_barrier()


            def dstep(j4, carry):
                ja0 = j4 * NQ * B

                @pl.when(ja0 < end)
                def _():
                    gd = []
                    for q in range(NQ):
                        gd.append(pltpu.async_copy(
                            x_hbm.at[:, pl.ds(g * F, F)]
                                 .at[srcall.at[pl.ds(ja0 + q * B, B)]],
                            rows[q], gsem[q]))
                        for i in range(B // LANES):
                            dstv[q][pl.ds(i * LANES, LANES)] = \
                                dstall[pl.ds(ja0 + q * B + i * LANES, LANES)]
                    sd = []
                    for q in range(NQ):
                        gd[q].wait()
                        sd.append(pltpu.async_copy(rows[q], acc.at[dstv[q]],
                                                   ssem[q], add=True))
                    for q in range(NQ):
                        sd[q].wait()
                return carry

            lax.fori_loop(0, nb4, dstep, 0)
            plsc.subcore_barrier()

            gbase = nbase + rbase

            @pl.when(gbase + RPT <= N)
            def _():
                pltpu.sync_copy(acc.at[pl.ds(rbase, RPT)],
                                agg_hbm.at[g].at[pl.ds(gbase, RPT)])

            @pl.when(gbase == N - 80)
            def _():
                pltpu.sync_copy(acc.at[pl.ds(rbase, 80)],
                                agg_hbm.at[g].at[pl.ds(gbase, 80)])

    return k(x_f, ei_flat, zrows)


_BN = 1000
_G1 = N // _BN


def _mlp1_body(xr, agt, ep, w1, b1, h1o, s1o, ss1o):
    sc = 1.0 + ep[0, 0]
    qpd = D // F
    ssum = jnp.zeros((1, H), jnp.float32)
    ssq = jnp.zeros((1, H), jnp.float32)
    for s in range(S):
        aggs = jnp.concatenate([agt[qpd * s + q] for q in range(qpd)], axis=1)
        hs = sc * xr[:, s, :] + aggs
        acc = jnp.dot(hs, w1[...], preferred_element_type=jnp.float32)
        acc = acc + b1[...]
        h1o[:, s, :] = acc
        ssum = ssum + jnp.sum(acc, axis=0, keepdims=True)
        ssq = ssq + jnp.sum(acc * acc, axis=0, keepdims=True)
    i = pl.program_id(0)

    @pl.when(i == 0)
    def _():
        s1o[...] = ssum
        ss1o[...] = ssq

    @pl.when(i != 0)
    def _():
        s1o[...] = s1o[...] + ssum
        ss1o[...] = ss1o[...] + ssq


def _mlp1(x_nat, agg_t, epsv, W1, b1):
    return pl.pallas_call(
        _mlp1_body,
        grid=(_G1,),
        in_specs=[
            pl.BlockSpec((_BN, S, D), lambda i: (i, 0, 0)),
            pl.BlockSpec((CH, _BN, F), lambda i: (0, i, 0)),
            pl.BlockSpec(memory_space=pltpu.SMEM),
            pl.BlockSpec((D, H), lambda i: (0, 0)),
            pl.BlockSpec((1, H), lambda i: (0, 0)),
        ],
        out_specs=[
            pl.BlockSpec((_BN, S, H), lambda i: (i, 0, 0)),
            pl.BlockSpec((1, H), lambda i: (0, 0)),
            pl.BlockSpec((1, H), lambda i: (0, 0)),
        ],
        out_shape=[
            jax.ShapeDtypeStruct((N, S, H), jnp.float32),
            jax.ShapeDtypeStruct((1, H), jnp.float32),
            jax.ShapeDtypeStruct((1, H), jnp.float32),
        ],
    )(x_nat, agg_t, epsv, W1, b1)


def _mlp2_body(h1, scl, sft, w2, b2, h2o, s2o, ss2o):
    t = jnp.maximum(h1[...] * scl[...] + sft[...], 0.0)
    t2 = t.reshape(_BN * S, H)
    acc = jnp.dot(t2, w2[...], preferred_element_type=jnp.float32) + b2[...]
    h2o[...] = acc.reshape(_BN, S, EMB)
    ssum = jnp.sum(acc, axis=0, keepdims=True)
    ssq = jnp.sum(acc * acc, axis=0, keepdims=True)
    i = pl.program_id(0)

    @pl.when(i == 0)
    def _():
        s2o[...] = ssum
        ss2o[...] = ssq

    @pl.when(i != 0)
    def _():
        s2o[...] = s2o[...] + ssum
        ss2o[...] = ss2o[...] + ssq


def _mlp2(h1, scale1, shift1, W2, b2):
    return pl.pallas_call(
        _mlp2_body,
        grid=(_G1,),
        in_specs=[
            pl.BlockSpec((_BN, S, H), lambda i: (i, 0, 0)),
            pl.BlockSpec((1, 1, H), lambda i: (0, 0, 0)),
            pl.BlockSpec((1, 1, H), lambda i: (0, 0, 0)),
            pl.BlockSpec((H, EMB), lambda i: (0, 0)),
            pl.BlockSpec((1, EMB), lambda i: (0, 0)),
        ],
        out_specs=[
            pl.BlockSpec((_BN, S, EMB), lambda i: (i, 0, 0)),
            pl.BlockSpec((1, EMB), lambda i: (0, 0)),
            pl.BlockSpec((1, EMB), lambda i: (0, 0)),
        ],
        out_shape=[
            jax.ShapeDtypeStruct((N, S, EMB), jnp.float32),
            jax.ShapeDtypeStruct((1, EMB), jnp.float32),
            jax.ShapeDtypeStruct((1, EMB), jnp.float32),
        ],
    )(h1, scale1, shift1, W2, b2)


_BN3 = 2000
_G3 = N // _BN3


def _mlp3_body(h2, scl, sft, out):
    out[...] = jnp.maximum(h2[...] * scl[...] + sft[...], 0.0)


def _mlp3(h2, scale2, shift2):
    return pl.pallas_call(
        _mlp3_body,
        grid=(_G3,),
        in_specs=[
            pl.BlockSpec((_BN3, S, EMB), lambda i: (i, 0, 0)),
            pl.BlockSpec((1, 1, EMB), lambda i: (0, 0, 0)),
            pl.BlockSpec((1, 1, EMB), lambda i: (0, 0, 0)),
        ],
        out_specs=pl.BlockSpec((_BN3, S, EMB), lambda i: (i, 0, 0)),
        out_shape=jax.ShapeDtypeStruct((N, S, EMB), jnp.float32),
    )(h2, scale2, shift2)


def _bn_scale_shift(ssum, ssq, gamma, beta, count):
    mean = ssum[0] / count
    var = ssq[0] / count - mean * mean
    inv = gamma / jnp.sqrt(var + 1e-5)
    return inv, (beta - mean * inv)


def kernel(x, edge_index, eps, W1, b1, gamma1, beta1, W2, b2, gamma2, beta2):
    x_f = x.reshape(N, CH * F)
    zrows = jnp.zeros((RPT, F), jnp.float32)
    agg_t = _sc_agg(x_f, edge_index.reshape(2 * E), zrows)

    epsv = eps.reshape(1, 1)
    h1, s1, ss1 = _mlp1(x, agg_t, epsv, W1, b1.reshape(1, H))
    cnt = float(N * S)
    scale1, shift1 = _bn_scale_shift(s1, ss1, gamma1, beta1, cnt)
    h2, s2, ss2 = _mlp2(h1, scale1.reshape(1, 1, H), shift1.reshape(1, 1, H),
                        W2, b2.reshape(1, EMB))
    scale2, shift2 = _bn_scale_shift(s2, ss2, gamma2, beta2, cnt)
    out = _mlp3(h2, scale2.reshape(1, 1, EMB), shift2.reshape(1, 1, EMB))
    return out


# speedup vs baseline: 22.9903x; 22.9903x over previous
import functools

import jax
import jax.numpy as jnp
from jax import lax
from jax.experimental import pallas as pl
from jax.experimental.pallas import tpu as pltpu
from jax.experimental.pallas import tpu_sc as plsc

N = 10000
S = 4
D = 256
E = 160000
H = 512
EMB = 256

F = 128
CH = (S * D) // F
NC = 2
NS = 16
EPT = E // NS
HN = 5120
ACCR = HN + 512
RPT = HN // NS
B = 80
NB = EPT // B
NQ = 2
LANES = 16


def _sc_agg(x_f, ei_flat, zrows):
    mesh = plsc.VectorSubcoreMesh(core_axis_name="c", subcore_axis_name="s",
                                  num_cores=NC)

    @functools.partial(
        pl.kernel,
        out_type=jax.ShapeDtypeStruct((CH, N, F), jnp.float32),
        mesh=mesh,
        scratch_types=(
            [pltpu.VMEM((EPT + NQ * B, ), jnp.int32)] * 2
            + [pltpu.VMEM((B,), jnp.int32)] * NQ
            + [pltpu.VMEM((B, F), jnp.float32)] * NQ
            + [pltpu.VMEM((RPT, F), jnp.float32)]
            + [pltpu.VMEM_SHARED((ACCR, F), jnp.float32)]
            + [pltpu.SemaphoreType.DMA] * (2 * NQ)
        ),
    )
    def k(x_hbm, ei_hbm, z_hbm, agg_hbm,
          srcall, dstall, d0, d1, r0, r1, zbuf, acc,
          g0, g1, s0, s1):
        dstv = [d0, d1]
        rows = [r0, r1]
        gsem = [g0, g1]
        ssem = [s0, s1]
        c = lax.axis_index("c")
        s = lax.axis_index("s")
        nbase = c * HN
        ebase = s * EPT
        rbase = s * RPT
        pltpu.sync_copy(z_hbm, zbuf)
        pltpu.sync_copy(ei_hbm.at[pl.ds(ebase, EPT)],
                        srcall.at[pl.ds(0, EPT)])
        pltpu.sync_copy(ei_hbm.at[pl.ds(E + ebase, EPT)],
                        dstall.at[pl.ds(0, EPT)])

        lane = lax.iota(jnp.int32, LANES)

        def comp(i, carry):
            dv = dstall[pl.ds(i * LANES, LANES)] - nbase
            ok = (dv >= 0) & (dv < HN)
            dstall[pl.ds(i * LANES, LANES)] = jnp.where(
                ok, dv, HN + ((dv + nbase) & 511))
            return carry

        lax.fori_loop(0, EPT // LANES, comp, 0)
        cnt = EPT

        for i in range((NQ * B) // LANES):
            off = EPT + i * LANES
            srcall[pl.ds(off, LANES)] = jnp.zeros((LANES,), jnp.int32)
            dstall[pl.ds(off, LANES)] = HN + (lax.iota(jnp.int32, LANES)
                                              + i * LANES)

        nb4 = (EPT + NQ * B - 1) // (NQ * B)
        end = ((cnt + NQ * B - 1) // (NQ * B)) * (NQ * B)

        for g in range(CH):
            pltpu.sync_copy(zbuf, acc.at[pl.ds(rbase, RPT)])
            plsc.subcore_barrier()


            def dstep(j4, carry):
                ja0 = j4 * NQ * B

                @pl.when(ja0 < end)
                def _():
                    gd = []
                    for q in range(NQ):
                        gd.append(pltpu.async_copy(
                            x_hbm.at[:, pl.ds(g * F, F)]
                                 .at[srcall.at[pl.ds(ja0 + q * B, B)]],
                            rows[q], gsem[q]))
                        for i in range(B // LANES):
                            dstv[q][pl.ds(i * LANES, LANES)] = \
                                dstall[pl.ds(ja0 + q * B + i * LANES, LANES)]
                    sd = []
                    for q in range(NQ):
                        gd[q].wait()
                        sd.append(pltpu.async_copy(rows[q], acc.at[dstv[q]],
                                                   ssem[q], add=True))
                    for q in range(NQ):
                        sd[q].wait()
                return carry

            lax.fori_loop(0, nb4, dstep, 0)
            plsc.subcore_barrier()

            gbase = nbase + rbase

            @pl.when(gbase + RPT <= N)
            def _():
                pltpu.sync_copy(acc.at[pl.ds(rbase, RPT)],
                                agg_hbm.at[g].at[pl.ds(gbase, RPT)])

            @pl.when(gbase == N - 80)
            def _():
                pltpu.sync_copy(acc.at[pl.ds(rbase, 80)],
                                agg_hbm.at[g].at[pl.ds(gbase, 80)])

    return k(x_f, ei_flat, zrows)


_BN = 1000
_G1 = N // _BN


def _mlp1_body(xr, agt, ep, w1, b1, h1o, s1o, ss1o):
    sc = 1.0 + ep[0, 0]
    qpd = D // F
    ssum = jnp.zeros((1, H), jnp.float32)
    ssq = jnp.zeros((1, H), jnp.float32)
    for s in range(S):
        aggs = jnp.concatenate([agt[qpd * s + q] for q in range(qpd)], axis=1)
        hs = sc * xr[:, s, :] + aggs
        acc = jnp.dot(hs, w1[...], preferred_element_type=jnp.float32)
        acc = acc + b1[...]
        h1o[:, s, :] = acc
        ssum = ssum + jnp.sum(acc, axis=0, keepdims=True)
        ssq = ssq + jnp.sum(acc * acc, axis=0, keepdims=True)
    i = pl.program_id(0)

    @pl.when(i == 0)
    def _():
        s1o[...] = ssum
        ss1o[...] = ssq

    @pl.when(i != 0)
    def _():
        s1o[...] = s1o[...] + ssum
        ss1o[...] = ss1o[...] + ssq


def _mlp1(x_nat, agg_t, epsv, W1, b1):
    return pl.pallas_call(
        _mlp1_body,
        grid=(_G1,),
        in_specs=[
            pl.BlockSpec((_BN, S, D), lambda i: (i, 0, 0)),
            pl.BlockSpec((CH, _BN, F), lambda i: (0, i, 0)),
            pl.BlockSpec(memory_space=pltpu.SMEM),
            pl.BlockSpec((D, H), lambda i: (0, 0)),
            pl.BlockSpec((1, H), lambda i: (0, 0)),
        ],
        out_specs=[
            pl.BlockSpec((_BN, S, H), lambda i: (i, 0, 0)),
            pl.BlockSpec((1, H), lambda i: (0, 0)),
            pl.BlockSpec((1, H), lambda i: (0, 0)),
        ],
        out_shape=[
            jax.ShapeDtypeStruct((N, S, H), jnp.float32),
            jax.ShapeDtypeStruct((1, H), jnp.float32),
            jax.ShapeDtypeStruct((1, H), jnp.float32),
        ],
    )(x_nat, agg_t, epsv, W1, b1)


def _mlp2_body(h1, scl, sft, w2, b2, h2o, s2o, ss2o):
    t = jnp.maximum(h1[...] * scl[...] + sft[...], 0.0)
    t2 = t.reshape(_BN * S, H)
    acc = jnp.dot(t2, w2[...], preferred_element_type=jnp.float32) + b2[...]
    h2o[...] = acc.reshape(_BN, S, EMB)
    ssum = jnp.sum(acc, axis=0, keepdims=True)
    ssq = jnp.sum(acc * acc, axis=0, keepdims=True)
    i = pl.program_id(0)

    @pl.when(i == 0)
    def _():
        s2o[...] = ssum
        ss2o[...] = ssq

    @pl.when(i != 0)
    def _():
        s2o[...] = s2o[...] + ssum
        ss2o[...] = ss2o[...] + ssq


def _mlp2(h1, scale1, shift1, W2, b2):
    return pl.pallas_call(
        _mlp2_body,
        grid=(_G1,),
        in_specs=[
            pl.BlockSpec((_BN, S, H), lambda i: (i, 0, 0)),
            pl.BlockSpec((1, 1, H), lambda i: (0, 0, 0)),
            pl.BlockSpec((1, 1, H), lambda i: (0, 0, 0)),
            pl.BlockSpec((H, EMB), lambda i: (0, 0)),
            pl.BlockSpec((1, EMB), lambda i: (0, 0)),
        ],
        out_specs=[
            pl.BlockSpec((_BN, S, EMB), lambda i: (i, 0, 0)),
            pl.BlockSpec((1, EMB), lambda i: (0, 0)),
            pl.BlockSpec((1, EMB), lambda i: (0, 0)),
        ],
        out_shape=[
            jax.ShapeDtypeStruct((N, S, EMB), jnp.float32),
            jax.ShapeDtypeStruct((1, EMB), jnp.float32),
            jax.ShapeDtypeStruct((1, EMB), jnp.float32),
        ],
    )(h1, scale1, shift1, W2, b2)


_BN3 = 2000
_G3 = N // _BN3


def _mlp3_body(h2, scl, sft, out):
    out[...] = jnp.maximum(h2[...] * scl[...] + sft[...], 0.0)


def _mlp3(h2, scale2, shift2):
    return pl.pallas_call(
        _mlp3_body,
        grid=(_G3,),
        in_specs=[
            pl.BlockSpec((_BN3, S, EMB), lambda i: (i, 0, 0)),
            pl.BlockSpec((1, 1, EMB), lambda i: (0, 0, 0)),
            pl.BlockSpec((1, 1, EMB), lambda i: (0, 0, 0)),
        ],
        out_specs=pl.BlockSpec((_BN3, S, EMB), lambda i: (i, 0, 0)),
        out_shape=jax.ShapeDtypeStruct((N, S, EMB), jnp.float32),
    )(h2, scale2, shift2)


def _bn_scale_shift(ssum, ssq, gamma, beta, count):
    mean = ssum[0] / count
    var = ssq[0] / count - mean * mean
    inv = gamma / jnp.sqrt(var + 1e-5)
    return inv, (beta - mean * inv)


def kernel(x, edge_index, eps, W1, b1, gamma1, beta1, W2, b2, gamma2, beta2):
    x_f = x.reshape(N, CH * F)
    zrows = jnp.zeros((RPT, F), jnp.float32)
    agg_t = _sc_agg(x_f, edge_index.reshape(2 * E), zrows)

    epsv = eps.reshape(1, 1)
    h1, s1, ss1 = _mlp1(x, agg_t, epsv, W1, b1.reshape(1, H))
    cnt = float(N * S)
    scale1, shift1 = _bn_scale_shift(s1, ss1, gamma1, beta1, cnt)
    h2, s2, ss2 = _mlp2(h1, scale1.reshape(1, 1, H), shift1.reshape(1, 1, H),
                        W2, b2.reshape(1, EMB))
    scale2, shift2 = _bn_scale_shift(s2, ss2, gamma2, beta2, cnt)
    out = _mlp3(h2, scale2.reshape(1, 1, EMB), shift2.reshape(1, 1, EMB))
    return out
